# baseline (speedup 1.0000x reference)
import functools

import jax
import jax.numpy as jnp
from jax import lax
from jax.experimental import pallas as pl
from jax.experimental.pallas import tpu as pltpu
from jax.experimental.pallas import tpu_sc as plsc

_B, _L, _F, _C = 64, 1024, 256, 2
_H, _O = 512, 1
_K = 64
_N = _K + 1
_R = 72
_SB = 8


def _select_kernel(cx_ref, cy_ref, len_ref, idx_ref):
    cx = cx_ref[...]
    cy = cy_ref[...]
    dx = cx - cx[:, 0:1]
    dy = cy - cy[:, 0:1]
    d2 = dx * dx + dy * dy
    col = lax.broadcasted_iota(jnp.int32, (_B, _L), 1)
    lens = len_ref[...]
    valid = (col >= 1) & (col < lens)
    inf = jnp.float32(jnp.inf)
    d2 = jnp.where(valid, d2, inf)
    big = jnp.int32(_L)
    colk = lax.broadcasted_iota(jnp.int32, (_B, _R), 1)

    def body(t, carry):
        d2c, idxb = carry
        mval = jnp.min(d2c, axis=1, keepdims=True)
        cand = d2c == mval
        ii = jnp.where(cand, col, big)
        midx = jnp.min(ii, axis=1, keepdims=True)
        pick = col == midx
        d2c = jnp.where(pick, inf, d2c)
        idxb = jnp.where(colk == t + 1, midx, idxb)
        return d2c, idxb

    idx0 = jnp.zeros((_B, _R), jnp.int32)
    _, idxf = lax.fori_loop(0, _K, body, (d2, idx0))
    row = lax.broadcasted_iota(jnp.int32, (_B, _R), 0)
    idx_ref[...] = idxf + row * _L


def _make_gather():
    info = plsc.get_sparse_core_info()
    nc, ns = info.num_cores, info.num_subcores
    nw = nc * ns
    n_rows = _B * _R
    b_per_w = n_rows // nw
    mesh = plsc.VectorSubcoreMesh(core_axis_name="c", subcore_axis_name="s")

    @functools.partial(
        pl.kernel, mesh=mesh,
        out_type=jax.ShapeDtypeStruct((n_rows, _F), jnp.float32),
        scratch_types=[
            pltpu.VMEM((b_per_w,), jnp.int32),
            pltpu.VMEM((b_per_w, _F), jnp.float32),
            pltpu.SemaphoreType.DMA,
        ],
    )
    def gather(table_hbm, idx_hbm, out_hbm, idx_v, rows_v, sem):
        wid = lax.axis_index("s") * nc + lax.axis_index("c")
        base = wid * b_per_w
        pltpu.sync_copy(idx_hbm.at[pl.ds(base, b_per_w)], idx_v)
        pltpu.async_copy(table_hbm.at[idx_v], rows_v, sem).wait()
        pltpu.sync_copy(rows_v, out_hbm.at[pl.ds(base, b_per_w)])

    return gather


def _dense_kernel(x_ref, wl1_ref, wr1_ref, bl1_ref, w2l_ref, w2r_ref,
                  bl2_ref, out_ref):
    x3 = x_ref[...]
    rowi = lax.broadcasted_iota(jnp.int32, (_SB, _R, 1), 1)
    node = (rowi < _N).astype(jnp.float32)
    sum_x = jnp.sum(x3 * node, axis=1)
    m1 = jnp.dot(sum_x * (1.0 / _N), wl1_ref[...],
                 preferred_element_type=jnp.float32) + bl1_ref[...]
    g = jnp.dot(x3.reshape(_SB * _R, _F), wr1_ref[...],
                preferred_element_type=jnp.float32)
    h3 = jnp.maximum(g.reshape(_SB, _R, _H) + m1[:, None, :], 0.0)
    t = jnp.sum(h3 * w2l_ref[...][None], axis=2, keepdims=True)
    tsum = jnp.sum(t * node, axis=1)
    h0 = h3[:, 0, :]
    h0w = jnp.sum(h0 * w2r_ref[...], axis=1, keepdims=True)
    out_ref[...] = tsum * (1.0 / _N) + h0w + bl2_ref[...]


def kernel(inputs, coords, targets, input_lengths, Wl1, bl1, Wr1, Wl2, bl2,
           Wr2):
    cx = coords[:, :, 0]
    cy = coords[:, :, 1]
    lens = input_lengths[:, None].astype(jnp.int32)
    idx = pl.pallas_call(
        _select_kernel,
        out_shape=jax.ShapeDtypeStruct((_B, _R), jnp.int32),
    )(cx, cy, lens)

    table = inputs.reshape(_B * _L, _F)
    x_gat = _make_gather()(table, idx.reshape(_B * _R))

    out = pl.pallas_call(
        _dense_kernel,
        grid=(_B // _SB,),
        in_specs=[
            pl.BlockSpec((_SB, _R, _F), lambda s: (s, 0, 0)),
            pl.BlockSpec((_F, _H), lambda s: (0, 0)),
            pl.BlockSpec((_F, _H), lambda s: (0, 0)),
            pl.BlockSpec((1, _H), lambda s: (0, 0)),
            pl.BlockSpec((1, _H), lambda s: (0, 0)),
            pl.BlockSpec((1, _H), lambda s: (0, 0)),
            pl.BlockSpec((1, 1), lambda s: (0, 0)),
        ],
        out_specs=pl.BlockSpec((_SB, 1), lambda s: (s, 0)),
        out_shape=jax.ShapeDtypeStruct((_B, 1), jnp.float32),
    )(x_gat.reshape(_B, _R, _F), Wl1.T, Wr1.T, bl1[None, :], Wl2, Wr2,
      bl2[None, :])

    target_head = targets[:, 0, :]
    return out, target_head
